# baseline (speedup 1.0000x reference)
import functools

import jax
import jax.numpy as jnp
from jax import lax
from jax.experimental import pallas as pl
from jax.experimental.pallas import tpu as pltpu
from jax.experimental.pallas import tpu_sc as plsc

B = 16384
H = 64
W = 2 * H
NE = 1000000
NR = 1000
L = 16
NC = 2
NS = 16
NW = NC * NS
BPW = B // NW
CB = 64
NCHUNK = BPW // CB
NBUF = 2
NGROUP = CB // L
KH = H // L


def _body(src_h, rel_h, dst_h, er_h, ei_h, rr_h, ri_h, out_h,
          s_raw, r_raw, d_raw, s_row, r_row, d_row,
          sr_b, si_b, dr_b, di_b, qr_b, qi_b,
          p_v, out_v, sem0, sem1):
    sems = (sem0, sem1)
    wid = lax.axis_index("s") * NC + lax.axis_index("c")
    base = wid * BPW

    stage = []
    for c in range(NCHUNK):
        off = base + c * CB
        stage.append(pltpu.async_copy(src_h.at[pl.ds(off, CB)], s_raw.at[c], sem0))
        stage.append(pltpu.async_copy(rel_h.at[pl.ds(off, CB)], r_raw.at[c], sem0))
        stage.append(pltpu.async_copy(dst_h.at[pl.ds(off, CB)], d_raw.at[c], sem0))
    for cp in stage:
        cp.wait()

    for c in range(NCHUNK):
        for g in range(NGROUP):
            ds = pl.ds(g * L, L)
            s_row[c, ds] = lax.shift_right_logical(s_raw[c, ds], 1)
            r_row[c, ds] = lax.shift_right_logical(r_raw[c, ds], 1)
            d_row[c, ds] = lax.shift_right_logical(d_raw[c, ds], 1)

    gathers = ((er_h, s_row, sr_b), (ei_h, s_row, si_b),
               (er_h, d_row, dr_b), (ei_h, d_row, di_b),
               (rr_h, r_row, qr_b), (ri_h, r_row, qi_b))

    def issue(cc, slot):
        for tab, rref, buf in gathers:
            pltpu.async_copy(tab.at[rref.at[cc]], buf.at[slot], sems[slot])

    def drain(cc, slot):
        for tab, rref, buf in gathers:
            pltpu.make_async_copy(tab.at[rref.at[cc]], buf.at[slot],
                                  sems[slot]).wait()

    def compute(cc, slot):
        def g_body(g, _):
            sv = s_raw[cc, pl.ds(g * L, L)]
            rv = r_raw[cc, pl.ds(g * L, L)]
            dv = d_raw[cc, pl.ds(g * L, L)]
            for el in range(L):
                e = g * L + el
                so = (sv[el] & 1) * H
                ro = (rv[el] & 1) * H
                do = (dv[el] & 1) * H
                acc = jnp.zeros((L,), jnp.float32)
                for k in range(KH):
                    a = sr_b[slot, e, pl.ds(so + k * L, L)]
                    bi = si_b[slot, e, pl.ds(so + k * L, L)]
                    cr = dr_b[slot, e, pl.ds(do + k * L, L)]
                    ci = di_b[slot, e, pl.ds(do + k * L, L)]
                    rr = qr_b[slot, e, pl.ds(ro + k * L, L)]
                    ri = qi_b[slot, e, pl.ds(ro + k * L, L)]
                    acc = acc + rr * (a * cr + bi * ci) + ri * (a * ci - bi * cr)
                p_v[pl.ds(el * (L + 1), L)] = acc
            rows = lax.iota(jnp.int32, L) * (L + 1)
            tot = jnp.zeros((L,), jnp.float32)
            for j in range(L):
                tot = tot + plsc.load_gather(p_v, [rows + j])
            out_v[pl.ds(cc * CB + g * L, L)] = tot
            return 0

        lax.fori_loop(0, NGROUP, g_body, 0)

    issue(0, 0)

    def pipe_body(it, _):
        for b in range(NBUF):
            cc = it * NBUF + b
            nxt = cc + 1

            @pl.when(nxt < NCHUNK)
            def _():
                issue(nxt, (b + 1) % NBUF)

            drain(cc, b)
            compute(cc, b)
        return 0

    lax.fori_loop(0, NCHUNK // NBUF, pipe_body, 0)

    pltpu.sync_copy(out_v, out_h.at[pl.ds(base, BPW)])


_sc_call = functools.partial(
    pl.kernel,
    out_type=jax.ShapeDtypeStruct((B,), jnp.float32),
    mesh=plsc.VectorSubcoreMesh(core_axis_name="c", subcore_axis_name="s"),
    compiler_params=pltpu.CompilerParams(needs_layout_passes=False),
    scratch_types=[
        pltpu.VMEM((NCHUNK, CB), jnp.int32),
        pltpu.VMEM((NCHUNK, CB), jnp.int32),
        pltpu.VMEM((NCHUNK, CB), jnp.int32),
        pltpu.VMEM((NCHUNK, CB), jnp.int32),
        pltpu.VMEM((NCHUNK, CB), jnp.int32),
        pltpu.VMEM((NCHUNK, CB), jnp.int32),
        pltpu.VMEM((NBUF, CB, W), jnp.float32),
        pltpu.VMEM((NBUF, CB, W), jnp.float32),
        pltpu.VMEM((NBUF, CB, W), jnp.float32),
        pltpu.VMEM((NBUF, CB, W), jnp.float32),
        pltpu.VMEM((NBUF, CB, W), jnp.float32),
        pltpu.VMEM((NBUF, CB, W), jnp.float32),
        pltpu.VMEM((L * (L + 1),), jnp.float32),
        pltpu.VMEM((BPW,), jnp.float32),
        pltpu.SemaphoreType.DMA,
        pltpu.SemaphoreType.DMA,
    ],
)(_body)


@jax.jit
def kernel(src, rel, dst, ent_real, ent_imag, rel_real, rel_imag):
    return _sc_call(src.astype(jnp.int32), rel.astype(jnp.int32),
                    dst.astype(jnp.int32),
                    ent_real.reshape(NE // 2, W),
                    ent_imag.reshape(NE // 2, W),
                    rel_real.reshape(NR // 2, W),
                    rel_imag.reshape(NR // 2, W))
